# Initial kernel scaffold; baseline (speedup 1.0000x reference)
import jax
import jax.numpy as jnp
from jax.experimental import pallas as pl


def kernel(x, edge_index, batch, W1, b1, W2, b2, W3, b3, Wl, bl):
    raise NotImplementedError("write your pallas kernel here")


# speedup vs baseline: 15.3369x; 15.3369x over previous
import functools

import jax
import jax.numpy as jnp
from jax import lax
from jax.experimental import pallas as pl
from jax.experimental.pallas import tpu as pltpu
from jax.experimental.pallas import tpu_sc as plsc

N = 10000
E = 320000
G = 64
NC = 2
NS = 16
NW = NC * NS
CHUNK = 128
NBUF = 4
EPW = 10112
NCHUNK = EPW // CHUNK
E_PAD = EPW * NW
ROWS_ACC = 10240
RPW = ROWS_ACC // NS
PAD_DST = N

@functools.lru_cache(maxsize=None)
def _mesh():
    return plsc.VectorSubcoreMesh(core_axis_name="c", subcore_axis_name="s",
                                  num_cores=NC, num_subcores=NS)


def _agg_body(nbuf, zeros_h, table_h, src_h, dst_h, out_h, acc, rows, sidx,
              didx, *sems):
    cid = lax.axis_index("c")
    sid = lax.axis_index("s")
    wid = sid * NC + cid
    ebase = wid * EPW

    pltpu.sync_copy(zeros_h, acc.at[pl.ds(sid * RPW, RPW)])
    plsc.subcore_barrier()

    def start(b, j):
        off = ebase + j * CHUNK
        pltpu.sync_copy(src_h.at[pl.ds(off, CHUNK)], sidx.at[b, 0])
        pltpu.sync_copy(dst_h.at[pl.ds(off, CHUNK)], didx.at[b, 0])
        return pltpu.async_copy(table_h.at[sidx.at[b, 0]], rows.at[b],
                                sems[b])

    def drain(b, d):
        d.wait()
        pltpu.sync_copy(rows.at[b], acc.at[didx.at[b, 0]], add=True)

    nq = NCHUNK // nbuf
    def group(i, carry):
        j0 = i * nbuf
        descs = [start(b, j0 + b) for b in range(nbuf)]
        for b in range(nbuf):
            drain(b, descs[b])
        return carry
    lax.fori_loop(0, nq, group, 0)
    tail = NCHUNK - nq * nbuf
    if tail:
        descs = [start(b, nq * nbuf + b) for b in range(tail)]
        for b in range(tail):
            drain(b, descs[b])

    plsc.subcore_barrier()
    pltpu.sync_copy(acc.at[pl.ds(sid * RPW, RPW)],
                    out_h.at[cid, pl.ds(sid * RPW, RPW)])


@functools.lru_cache(maxsize=None)
def _make_agg(w):
    nbuf = 4 if w <= 64 else 2
    return pl.kernel(
        functools.partial(_agg_body, nbuf),
        out_type=jax.ShapeDtypeStruct((NC, ROWS_ACC, w), jnp.float32),
        mesh=_mesh(),
        scratch_types=[
            pltpu.VMEM_SHARED((ROWS_ACC, w), jnp.float32),
            pltpu.VMEM((nbuf, CHUNK, w), jnp.float32),
            pltpu.VMEM((nbuf, 1, CHUNK), jnp.int32),
            pltpu.VMEM((nbuf, 1, CHUNK), jnp.int32),
        ] + [pltpu.SemaphoreType.DMA] * nbuf,
        compiler_params=pltpu.CompilerParams(use_tc_tiling_on_sc=False),
    )


def _deg_body(zeros_h, ones_h, dst_h, out_h, acc, ones_v, didx):
    cid = lax.axis_index("c")
    sid = lax.axis_index("s")
    wid = sid * NC + cid
    ebase = wid * EPW

    pltpu.sync_copy(zeros_h, acc.at[pl.ds(sid * RPW, RPW)])
    pltpu.sync_copy(ones_h, ones_v)
    plsc.subcore_barrier()

    def body(j, carry):
        off = ebase + j * CHUNK
        pltpu.sync_copy(dst_h.at[pl.ds(off, CHUNK)], didx.at[0, 0])
        pltpu.sync_copy(ones_v, acc.at[didx.at[0, 0]], add=True)
        return carry
    lax.fori_loop(0, NCHUNK, body, 0)

    plsc.subcore_barrier()
    pltpu.sync_copy(acc.at[pl.ds(sid * RPW, RPW)],
                    out_h.at[cid, pl.ds(sid * RPW, RPW)])


@functools.lru_cache(maxsize=None)
def _make_deg():
    return pl.kernel(
        _deg_body,
        out_type=jax.ShapeDtypeStruct((NC, ROWS_ACC, 16), jnp.float32),
        mesh=_mesh(),
        scratch_types=[
            pltpu.VMEM_SHARED((ROWS_ACC, 16), jnp.float32),
            pltpu.VMEM((CHUNK, 16), jnp.float32),
            pltpu.VMEM((1, 1, CHUNK), jnp.int32),
        ],
        compiler_params=pltpu.CompilerParams(use_tc_tiling_on_sc=False),
    )


def _k1_body(x_ref, w1_ref, d0_ref, d1_ref, h1_ref, g1_ref, dinv_ref):
    deg = d0_ref[:, 0] + d1_ref[:, 0] + 1.0
    dinv = lax.rsqrt(deg)
    h = jnp.dot(x_ref[...], w1_ref[...], preferred_element_type=jnp.float32,
                precision=lax.Precision.HIGHEST)
    h1_ref[...] = h
    g1_ref[...] = h * dinv[:, None]
    dinv_ref[...] = dinv[:, None]


def _k2_body(s0_ref, s1_ref, h1_ref, dinv_ref, b1_ref, x2_ref, g2_ref):
    dinv = dinv_ref[:, 0]
    z = (dinv[:, None] * (s0_ref[...] + s1_ref[...])
         + (dinv * dinv)[:, None] * h1_ref[...] + b1_ref[...])
    x2 = jnp.maximum(z, 0.0)
    x2_ref[...] = x2
    g2_ref[...] = x2 * dinv[:, None]


def _k3_body(s0_ref, s1_ref, x2_ref, dinv_ref, w2_ref, b2_ref,
             x3_ref, g3_ref):
    dinv = dinv_ref[:, 0]
    u = (dinv[:, None] * (s0_ref[...] + s1_ref[...])
         + (dinv * dinv)[:, None] * x2_ref[...])
    x3 = jnp.maximum(
        jnp.dot(u, w2_ref[...], preferred_element_type=jnp.float32,
                precision=lax.Precision.HIGHEST)
        + b2_ref[...], 0.0)
    x3_ref[...] = x3
    g3_ref[...] = x3 * dinv[:, None]


def _k4_body(s0_ref, s1_ref, x3_ref, dinv_ref, batch_ref, w3_ref, b3_ref,
             wl_ref, bl_ref, out_ref):
    dinv = dinv_ref[:, 0]
    z3 = (dinv[:, None] * (s0_ref[...] + s1_ref[...])
          + (dinv * dinv)[:, None] * x3_ref[...])
    bid = batch_ref[0, :]
    gid = lax.broadcasted_iota(jnp.int32, (G, N), 0)
    m = (bid[None, :] == gid).astype(jnp.float32)
    sums = jnp.dot(m, z3, preferred_element_type=jnp.float32,
                precision=lax.Precision.HIGHEST)
    cnt = jnp.sum(m, axis=1)
    pooled = sums / jnp.maximum(cnt, 1.0)[:, None]
    wf = jnp.dot(w3_ref[...], wl_ref[...], preferred_element_type=jnp.float32,
                precision=lax.Precision.HIGHEST)
    bf = jnp.dot(b3_ref[...], wl_ref[...], preferred_element_type=jnp.float32,
                precision=lax.Precision.HIGHEST)
    out_ref[...] = (jnp.dot(pooled, wf, preferred_element_type=jnp.float32,
                precision=lax.Precision.HIGHEST)
                    + bf + bl_ref[0, 0])


def _tc_call(body, out_shapes):
    return pl.pallas_call(body, out_shape=out_shapes)


def kernel(x, edge_index, batch, W1, b1, W2, b2, W3, b3, Wl, bl):
    src = edge_index[0].astype(jnp.int32)
    dst = edge_index[1].astype(jnp.int32)
    npad = E_PAD - E
    src_p = jnp.concatenate([src, jnp.zeros((npad,), jnp.int32)])
    dst_p = jnp.concatenate([dst, jnp.full((npad,), PAD_DST, jnp.int32)])
    zeros1 = jnp.zeros((RPW, 16), jnp.float32)
    ones1 = jnp.ones((CHUNK, 16), jnp.float32)
    zeros64 = jnp.zeros((RPW, 64), jnp.float32)
    zeros128 = jnp.zeros((RPW, 128), jnp.float32)

    deg2 = _make_deg()(zeros1, ones1, dst_p)
    d0 = deg2[0, :N, :1]
    d1 = deg2[1, :N, :1]

    h1, g1, dinv = _tc_call(_k1_body, [
        jax.ShapeDtypeStruct((N, 64), jnp.float32),
        jax.ShapeDtypeStruct((N, 64), jnp.float32),
        jax.ShapeDtypeStruct((N, 1), jnp.float32),
    ])(x, W1, d0, d1)

    s1 = _make_agg(64)(zeros64, g1, src_p, dst_p)
    x2, g2 = _tc_call(_k2_body, [
        jax.ShapeDtypeStruct((N, 64), jnp.float32),
        jax.ShapeDtypeStruct((N, 64), jnp.float32),
    ])(s1[0, :N], s1[1, :N], h1, dinv, b1[None, :])

    s2 = _make_agg(64)(zeros64, g2, src_p, dst_p)
    x3, g3 = _tc_call(_k3_body, [
        jax.ShapeDtypeStruct((N, 128), jnp.float32),
        jax.ShapeDtypeStruct((N, 128), jnp.float32),
    ])(s2[0, :N], s2[1, :N], x2, dinv, W2, b2[None, :])

    s3 = _make_agg(128)(zeros128, g3, src_p, dst_p)
    out = _tc_call(_k4_body, [
        jax.ShapeDtypeStruct((G, 1), jnp.float32),
    ])(s3[0, :N], s3[1, :N], x3, dinv, batch.astype(jnp.int32)[None, :],
       W3, b3[None, :], Wl, bl[None, :])[0]
    return out
